# baseline (speedup 1.0000x reference)
import jax
import jax.numpy as jnp
from jax.experimental import pallas as pl

_K = 20
_NP = 96
_H = 14
_W = 14
_U = 196
_R = 4


def _perm76():
    return jax.random.permutation(
        jax.random.key(1), _H * _W - _K)[: _NP - _K]


def _body(xp_ref, wt_ref, dwb_ref, wrow_ref, pwb_ref, perm_ref, x_ref, y_ref):
    xf = xp_ref[...].astype(jnp.float32)

    uu = jax.lax.broadcasted_iota(jnp.int32, (1, 1, _U), 2)
    jmod = jax.lax.rem(uu, _W)
    imod = jax.lax.div(uu, _W)
    wrap_lo = jmod == 0
    wrap_hi = jmod == _W - 1

    acc = None
    for dy in range(3):
        for dx in range(3):
            o = dy * _W + dx
            term = xf[:, :, o:o + _U] * wt_ref[3 * dy + dx][None, :, :]
            if dx == 0:
                term = jnp.where(wrap_lo, 0.0, term)
            elif dx == 2:
                term = jnp.where(wrap_hi, 0.0, term)
            acc = term if acc is None else acc + term
    acc = acc + dwb_ref[...][None, :, :]
    tb = acc.astype(jnp.bfloat16)

    rows = []
    for r in range(_R):
        o = jax.lax.dot_general(wrow_ref[...], tb[r], (((1,), (0,)), ((), ())),
                                preferred_element_type=jnp.float32)
        rows.append(o[0:1, :])
    p = jnp.concatenate(rows, axis=0) + pwb_ref[...][0:1, 0:1]
    s = 1.0 / (1.0 + jnp.exp(-p))

    sa = s[:, :, None]
    sb = s[:, None, :]
    aidx = jax.lax.broadcasted_iota(jnp.int32, (1, _U, 1), 1)
    less = uu < aidx
    m = (sb > sa) | ((sb == sa) & less)
    rank = jnp.sum(m.astype(jnp.float32), axis=2)

    unsel = rank >= float(_K)
    ur = jnp.sum((less & unsel[:, None, :]).astype(jnp.float32), axis=2)

    clipx = jnp.clip(jmod, 1, _W - 1).astype(jnp.float32)
    clipy = jnp.clip(imod, 1, _H - 1).astype(jnp.float32)

    karr = jax.lax.broadcasted_iota(jnp.int32, (1, _K, 1), 1).astype(jnp.float32)
    oh20 = (rank[:, None, :] == karr).astype(jnp.float32)
    x_sel = jnp.sum(oh20 * clipx, axis=2)
    y_sel = jnp.sum(oh20 * clipy, axis=2)

    perm3 = perm_ref[...][None, :, 0:1]
    oh76 = ((ur[:, None, :] == perm3) & unsel[:, None, :]).astype(jnp.float32)
    x_rem = jnp.sum(oh76 * clipx, axis=2)
    y_rem = jnp.sum(oh76 * clipy, axis=2)

    x_ref[...] = jnp.concatenate((x_sel, x_rem), axis=1).astype(jnp.int32)[None]
    y_ref[...] = jnp.concatenate((y_sel, y_rem), axis=1).astype(jnp.int32)[None]


def kernel(feature, dw_w, dw_b, pw_w, pw_b):
    b, n, c, h, w = feature.shape
    xb = feature[0].reshape(n, c, h * w).astype(jnp.bfloat16)
    xpad = jnp.pad(xb, ((0, 0), (0, 0), (15, 15)))
    wt9 = jnp.transpose(dw_w[:, 0], (1, 2, 0)).reshape(9, c)[:, :, None]
    dwb = dw_b[:, None]
    wrow = jnp.zeros((8, c), jnp.float32).at[0, :].set(pw_w[0, :, 0, 0])
    pwb = jnp.broadcast_to(pw_b[:, None], (8, 128))
    perm = _perm76().astype(jnp.float32)[:, None]

    f = pl.pallas_call(
        _body,
        grid=(n // _R,),
        in_specs=[pl.BlockSpec((_R, c, 226), lambda i: (i, 0, 0)),
                  pl.BlockSpec((9, c, 1), lambda i: (0, 0, 0)),
                  pl.BlockSpec((c, 1), lambda i: (0, 0)),
                  pl.BlockSpec((8, c), lambda i: (0, 0)),
                  pl.BlockSpec((8, 128), lambda i: (0, 0)),
                  pl.BlockSpec((_NP - _K, 1), lambda i: (0, 0))],
        out_specs=[pl.BlockSpec((1, _R, _NP), lambda i: (i, 0, 0)),
                   pl.BlockSpec((1, _R, _NP), lambda i: (i, 0, 0))],
        out_shape=[jax.ShapeDtypeStruct((n // _R, _R, _NP), jnp.int32),
                   jax.ShapeDtypeStruct((n // _R, _R, _NP), jnp.int32)],
    )
    x, y = f(xpad, wt9, dwb, wrow, pwb, perm)
    return (x.reshape(n, _NP), y.reshape(n, _NP))


# speedup vs baseline: 1.8056x; 1.8056x over previous
import jax
import jax.numpy as jnp
from jax.experimental import pallas as pl

_K = 20
_NP = 96
_H = 14
_W = 14
_U = 196
_R = 4


def _perm76():
    return jax.random.permutation(
        jax.random.key(1), _H * _W - _K)[: _NP - _K]


def _body(xp_ref, wt_ref, dwb_ref, wrow_ref, pwb_ref, perm_ref, coords_ref,
          x_ref, y_ref):
    xf = xp_ref[...].astype(jnp.float32)

    uu = jax.lax.broadcasted_iota(jnp.int32, (1, 1, _U), 2)
    jmod = jax.lax.rem(uu, _W)
    imod = jax.lax.div(uu, _W)
    wrap_lo = jmod == 0
    wrap_hi = jmod == _W - 1

    acc = None
    for dy in range(3):
        for dx in range(3):
            o = dy * _W + dx
            term = xf[:, :, o:o + _U] * wt_ref[3 * dy + dx][None, :, :]
            if dx == 0:
                term = jnp.where(wrap_lo, 0.0, term)
            elif dx == 2:
                term = jnp.where(wrap_hi, 0.0, term)
            acc = term if acc is None else acc + term
    acc = acc + dwb_ref[...][None, :, :]
    tb = acc.astype(jnp.bfloat16)

    rows = []
    for r in range(_R):
        o = jax.lax.dot_general(wrow_ref[...], tb[r], (((1,), (0,)), ((), ())),
                                preferred_element_type=jnp.float32)
        rows.append(o[0:1, :])
    p = jnp.concatenate(rows, axis=0) + pwb_ref[...][0:1, 0:1]
    s = 1.0 / (1.0 + jnp.exp(-p))

    sa = s[:, :, None]
    sb = s[:, None, :]
    aidx = jax.lax.broadcasted_iota(jnp.int32, (1, _U, 1), 1)
    less = uu < aidx
    m = (sb > sa) | ((sb == sa) & less)
    rank = jnp.sum(m.astype(jnp.float32), axis=2)

    unsel = rank >= float(_K)
    ur = jnp.sum((less & unsel[:, None, :]).astype(jnp.float32), axis=2)

    karr = jax.lax.broadcasted_iota(jnp.int32, (1, _K, 1), 1).astype(jnp.float32)
    oh20 = (rank[:, None, :] == karr).astype(jnp.float32)

    perm3 = perm_ref[...][None, :, 0:1]
    oh76 = ((ur[:, None, :] == perm3) & unsel[:, None, :]).astype(jnp.float32)

    ohcat = jnp.concatenate((oh20, oh76), axis=1).reshape(_R * _NP, _U)
    xy = jax.lax.dot_general(ohcat, coords_ref[...], (((1,), (0,)), ((), ())),
                             preferred_element_type=jnp.float32)
    x_ref[...] = xy[:, 0].astype(jnp.int32).reshape(1, _R, _NP)
    y_ref[...] = xy[:, 1].astype(jnp.int32).reshape(1, _R, _NP)


def kernel(feature, dw_w, dw_b, pw_w, pw_b):
    b, n, c, h, w = feature.shape
    xb = feature[0].reshape(n, c, h * w).astype(jnp.bfloat16)
    xpad = jnp.pad(xb, ((0, 0), (0, 0), (15, 15)))
    wt9 = jnp.transpose(dw_w[:, 0], (1, 2, 0)).reshape(9, c)[:, :, None]
    dwb = dw_b[:, None]
    wrow = jnp.zeros((8, c), jnp.float32).at[0, :].set(pw_w[0, :, 0, 0])
    pwb = jnp.broadcast_to(pw_b[:, None], (8, 128))
    perm = _perm76().astype(jnp.float32)[:, None]
    u = jnp.arange(_U)
    coords = jnp.zeros((_U, 128), jnp.float32)
    coords = coords.at[:, 0].set(jnp.clip(u % _W, 1, _W - 1).astype(jnp.float32))
    coords = coords.at[:, 1].set(jnp.clip(u // _W, 1, _H - 1).astype(jnp.float32))

    f = pl.pallas_call(
        _body,
        grid=(n // _R,),
        in_specs=[pl.BlockSpec((_R, c, 226), lambda i: (i, 0, 0)),
                  pl.BlockSpec((9, c, 1), lambda i: (0, 0, 0)),
                  pl.BlockSpec((c, 1), lambda i: (0, 0)),
                  pl.BlockSpec((8, c), lambda i: (0, 0)),
                  pl.BlockSpec((8, 128), lambda i: (0, 0)),
                  pl.BlockSpec((_NP - _K, 1), lambda i: (0, 0)),
                  pl.BlockSpec((_U, 128), lambda i: (0, 0))],
        out_specs=[pl.BlockSpec((1, _R, _NP), lambda i: (i, 0, 0)),
                   pl.BlockSpec((1, _R, _NP), lambda i: (i, 0, 0))],
        out_shape=[jax.ShapeDtypeStruct((n // _R, _R, _NP), jnp.int32),
                   jax.ShapeDtypeStruct((n // _R, _R, _NP), jnp.int32)],
    )
    x, y = f(xpad, wt9, dwb, wrow, pwb, perm, coords)
    return (x.reshape(n, _NP), y.reshape(n, _NP))
